# baseline (speedup 1.0000x reference)
import functools

import jax
import jax.numpy as jnp
from jax import lax
from jax.experimental import pallas as pl
from jax.experimental.pallas import tpu as pltpu
from jax.experimental.pallas import tpu_sc as plsc

BATCH = 16384
EMBED_DIM = 32
L = 16
NW = 32
B_PER_W = BATCH // NW
NCHUNK = 4
CHUNK = B_PER_W // NCHUNK


def _mf_body(user_ids, item_ids, user_table, item_table, out_hbm,
             idx_u, idx_i, rows_u, rows_i, out_v, sem):
    wid = lax.axis_index("s") * 2 + lax.axis_index("c")
    base = wid * B_PER_W

    for k in range(NCHUNK):
        pltpu.sync_copy(user_ids.at[pl.ds(base + k * CHUNK, CHUNK)], idx_u.at[k])
        pltpu.sync_copy(item_ids.at[pl.ds(base + k * CHUNK, CHUNK)], idx_i.at[k])
    for k in range(NCHUNK):
        for j in range(CHUNK // L):
            s = pl.ds(j * L, L)
            idx_u[k, s] = idx_u[k, s] + 1
            idx_i[k, s] = idx_i[k, s] + 1

    copies = []
    for k in range(NCHUNK):
        copies.append(pltpu.async_copy(
            user_table.at[idx_u.at[k]], rows_u.at[pl.ds(k * CHUNK, CHUNK)], sem))
        copies.append(pltpu.async_copy(
            item_table.at[idx_i.at[k]], rows_i.at[pl.ds(k * CHUNK, CHUNK)], sem))
    for c in copies:
        c.wait()

    iota = lax.iota(jnp.int32, L)

    def group_body(g, carry):
        row = g * L + iota
        acc = jnp.zeros((L,), jnp.float32)
        for d in range(EMBED_DIM):
            dvec = jnp.full((L,), d, jnp.int32)
            u = plsc.load_gather(rows_u, [row, dvec])
            v = plsc.load_gather(rows_i, [row, dvec])
            acc = acc + u * v
        out_v[pl.ds(g * L, L)] = acc
        return carry

    lax.fori_loop(0, B_PER_W // L, group_body, 0)

    pltpu.sync_copy(out_v, out_hbm.at[pl.ds(base, B_PER_W)])


@jax.jit
def _mf(user_ids, item_ids, user_table, item_table):
    mesh = plsc.VectorSubcoreMesh(core_axis_name="c", subcore_axis_name="s")
    return pl.kernel(
        _mf_body,
        out_type=jax.ShapeDtypeStruct((BATCH,), jnp.float32),
        mesh=mesh,
        compiler_params=pltpu.CompilerParams(
            needs_layout_passes=False, use_tc_tiling_on_sc=False),
        scratch_types=[
            pltpu.VMEM((NCHUNK, CHUNK), jnp.int32),
            pltpu.VMEM((NCHUNK, CHUNK), jnp.int32),
            pltpu.VMEM((B_PER_W, EMBED_DIM), jnp.float32),
            pltpu.VMEM((B_PER_W, EMBED_DIM), jnp.float32),
            pltpu.VMEM((B_PER_W,), jnp.float32),
            pltpu.SemaphoreType.DMA,
        ],
    )(user_ids, item_ids, user_table, item_table)


def kernel(user_ids, item_ids, user_table, item_table):
    return _mf(user_ids, item_ids, user_table, item_table)
